# baseline (speedup 1.0000x reference)
import functools

import jax
import jax.numpy as jnp
from jax import lax
from jax.experimental import pallas as pl
from jax.experimental.pallas import tpu as pltpu
from jax.experimental.pallas import tpu_sc as plsc

_NUM_CORES = 2
_NUM_SUBCORES = 16
_NUM_WORKERS = _NUM_CORES * _NUM_SUBCORES


def _pe_broadcast(pe, batch, seq_len, chunk):
    embed_dim = pe.shape[1]
    rows_per_w = seq_len // _NUM_WORKERS
    n_chunks = rows_per_w // chunk
    mesh = plsc.VectorSubcoreMesh(
        core_axis_name="c",
        subcore_axis_name="s",
        num_cores=_NUM_CORES,
        num_subcores=_NUM_SUBCORES,
    )

    @functools.partial(
        pl.kernel,
        out_type=jax.ShapeDtypeStruct((batch, seq_len, embed_dim), pe.dtype),
        mesh=mesh,
        scratch_types=[
            pltpu.VMEM((chunk, embed_dim), pe.dtype),
            pltpu.VMEM((chunk, embed_dim), pe.dtype),
            pltpu.SemaphoreType.DMA,
            pltpu.SemaphoreType.DMA,
            pltpu.SemaphoreType.DMA,
            pltpu.SemaphoreType.DMA,
        ],
    )
    def broadcast_kernel(pe_hbm, out_hbm, buf0, buf1, ld0, ld1, st0, st1):
        bufs, lds, sts = (buf0, buf1), (ld0, ld1), (st0, st1)
        wid = lax.axis_index("s") * _NUM_CORES + lax.axis_index("c")
        row0 = wid * rows_per_w

        def start_load(c):
            return pltpu.async_copy(
                pe_hbm.at[pl.ds(row0 + c * chunk, chunk)], bufs[c % 2], lds[c % 2]
            )

        def start_stores(c):
            return [
                pltpu.async_copy(
                    bufs[c % 2],
                    out_hbm.at[b, pl.ds(row0 + c * chunk, chunk)],
                    sts[c % 2],
                )
                for b in range(batch)
            ]

        loads, stores = {}, {}
        loads[0] = start_load(0)
        for c in range(n_chunks):
            if c + 1 < n_chunks:
                if c - 1 >= 0:
                    for h in stores[c - 1]:
                        h.wait()
                loads[c + 1] = start_load(c + 1)
            loads[c].wait()
            stores[c] = start_stores(c)
        for c in range(max(0, n_chunks - 2), n_chunks):
            for h in stores[c]:
                h.wait()

    return broadcast_kernel


def kernel(x, pe):
    batch, seq_len = x.shape[0], x.shape[1]
    return _pe_broadcast(pe, batch, seq_len, chunk=32)(pe)
